# Initial kernel scaffold; baseline (speedup 1.0000x reference)
import jax
import jax.numpy as jnp
from jax.experimental import pallas as pl


def kernel(x, edge_index, W_self, W_neigh, b, ln_w, ln_b):
    raise NotImplementedError("write your pallas kernel here")


# speedup vs baseline: 2.8960x; 2.8960x over previous
import dataclasses
import functools

import jax
import jax.numpy as jnp
from jax import lax
from jax.experimental import pallas as pl
from jax.experimental.pallas import tpu as pltpu
from jax.experimental.pallas import tpu_sc as plsc

N = 10000
D = 128
E = 320000
L = 3

NC = 2
NS = 16
NW = NC * NS
CH = 128
EDGES_PER_TILE = -(-E // NW)
NCHUNK = 2 * (-(-EDGES_PER_TILE // (2 * CH)))
EPT = NCHUNK * CH
EPAD = EPT * NW
SHIFT = 14
NP = NS * 640
DUMP = N
ROWS_PER_TILE = NP // NS

_mesh = plsc.VectorSubcoreMesh(core_axis_name="c", subcore_axis_name="s",
                               num_cores=NC, num_subcores=NS)


def _unpack_idx(pk_v, src_v, dst_v):
    @pl.loop(0, NCHUNK)
    def _(k):
        @pl.loop(0, CH // 16)
        def _(j):
            v = pk_v[k, pl.ds(j * 16, 16)]
            src_v[k, pl.ds(j * 16, 16)] = jnp.bitwise_and(v, (1 << SHIFT) - 1)
            dst_v[k, pl.ds(j * 16, 16)] = jnp.right_shift(v, SHIFT)


def _sc_body(h_hbm, pk_hbm, acc_hbm, pk_v, src_v, dst_v, rows0, rows1,
             acc_s, sem0, sem1):
    cid = lax.axis_index("c")
    sid = lax.axis_index("s")
    wid = cid * NS + sid

    @pl.loop(0, CH)
    def _(r):
        @pl.loop(0, D // 16)
        def _(c):
            rows0[r, pl.ds(c * 16, 16)] = jnp.zeros((16,), jnp.float32)

    row0 = sid * ROWS_PER_TILE

    @pl.loop(0, ROWS_PER_TILE // CH)
    def _(j):
        pltpu.sync_copy(rows0, acc_s.at[pl.ds(row0 + j * CH, CH)])

    plsc.subcore_barrier()

    pltpu.sync_copy(pk_hbm.at[wid], pk_v)
    _unpack_idx(pk_v, src_v, dst_v)

    @pl.loop(0, NCHUNK)
    def _(k):
        pltpu.async_copy(h_hbm.at[src_v.at[k]], rows0, sem0).wait()
        pltpu.sync_copy(rows0, acc_s.at[dst_v.at[k]], add=True)

    plsc.subcore_barrier()

    pltpu.sync_copy(acc_s.at[pl.ds(row0, ROWS_PER_TILE)],
                    acc_hbm.at[cid, pl.ds(row0, ROWS_PER_TILE)])


def _make_sc_kernel():
    scratch = [
        pltpu.VMEM((NCHUNK, CH), jnp.int32),
        pltpu.VMEM((NCHUNK, CH), jnp.int32),
        pltpu.VMEM((NCHUNK, CH), jnp.int32),
        pltpu.VMEM((CH, D), jnp.float32),
        pltpu.VMEM((CH, D), jnp.float32),
        pltpu.VMEM_SHARED((NP, D), jnp.float32),
        pltpu.SemaphoreType.DMA,
        pltpu.SemaphoreType.DMA,
    ]
    return pl.kernel(_sc_body,
                     out_type=[jax.ShapeDtypeStruct((NC, NP, D), jnp.float32)],
                     mesh=_mesh, scratch_types=scratch)


def _sc_deg_body(pk_hbm, deg_hbm, pk_v, hist_v):
    cid = lax.axis_index("c")
    sid = lax.axis_index("s")
    wid = cid * NS + sid

    @pl.loop(0, NP // 16)
    def _(j):
        hist_v[pl.ds(j * 16, 16)] = jnp.zeros((16,), jnp.float32)

    pltpu.sync_copy(pk_hbm.at[wid], pk_v)
    ones16 = jnp.ones((16,), jnp.float32)

    @pl.loop(0, EPT // 16)
    def _(j):
        v = pk_v[pl.ds(j * 16, 16)]
        plsc.addupdate_scatter(hist_v, [jnp.right_shift(v, SHIFT)], ones16)

    pltpu.sync_copy(hist_v, deg_hbm.at[wid])


def _make_sc_deg_kernel():
    scratch = [
        pltpu.VMEM((EPT,), jnp.int32),
        pltpu.VMEM((NP,), jnp.float32),
    ]
    cp = pltpu.CompilerParams()
    if "needs_layout_passes" in pltpu.CompilerParams.__dataclass_fields__:
        cp = dataclasses.replace(cp, needs_layout_passes=False)
    return pl.kernel(_sc_deg_body,
                     out_type=[jax.ShapeDtypeStruct((NW, NP), jnp.float32)],
                     mesh=_mesh, scratch_types=scratch,
                     compiler_params=cp)


def _tc_dense(acc, deg, h, w_self, w_neigh, bb, lnw, lnb):
    BR = 1000
    grid = (N // BR,)

    def body(acc_ref, deg_ref, h_ref, wn_ref, ws_ref, b_ref,
             lnw_ref, lnb_ref, o_ref):
        s = acc_ref[0] + acc_ref[1]
        dsum = jnp.sum(deg_ref[...], axis=1, keepdims=True)
        agg = s / jnp.maximum(dsum, 1.0)
        hh = h_ref[...]
        dn = (((1,), (1,)), ((), ()))
        out = lax.dot_general(agg, wn_ref[...], dn,
                              preferred_element_type=jnp.float32,
                              precision=lax.Precision.HIGHEST)
        out += lax.dot_general(hh, ws_ref[...], dn,
                               preferred_element_type=jnp.float32,
                               precision=lax.Precision.HIGHEST)
        out += b_ref[...]
        mu = jnp.mean(out, axis=-1, keepdims=True)
        xc = out - mu
        var = jnp.mean(xc * xc, axis=-1, keepdims=True)
        out = xc * lax.rsqrt(var + 1e-5) * lnw_ref[...] + lnb_ref[...]
        o_ref[...] = jnp.maximum(out, 0.0) + hh

    return pl.pallas_call(
        body,
        grid=grid,
        in_specs=[
            pl.BlockSpec((NC, BR, D), lambda i: (0, i, 0)),
            pl.BlockSpec((BR, NW), lambda i: (i, 0)),
            pl.BlockSpec((BR, D), lambda i: (i, 0)),
            pl.BlockSpec((D, D), lambda i: (0, 0)),
            pl.BlockSpec((D, D), lambda i: (0, 0)),
            pl.BlockSpec((1, D), lambda i: (0, 0)),
            pl.BlockSpec((1, D), lambda i: (0, 0)),
            pl.BlockSpec((1, D), lambda i: (0, 0)),
        ],
        out_specs=pl.BlockSpec((BR, D), lambda i: (i, 0)),
        out_shape=jax.ShapeDtypeStruct((N, D), jnp.float32),
    )(acc, deg, h, w_neigh, w_self, bb, lnw, lnb)


def kernel(x, edge_index, W_self, W_neigh, b, ln_w, ln_b):
    src = edge_index[0]
    dst = edge_index[1]
    pad = EPAD - E
    packed = jnp.bitwise_or(src, jnp.left_shift(dst, SHIFT))
    packed = jnp.concatenate(
        [packed, jnp.full((pad,), DUMP << SHIFT, jnp.int32)])
    pk3 = packed.reshape(NW, NCHUNK, CH)

    scl = _make_sc_kernel()
    (degp,) = _make_sc_deg_kernel()(packed.reshape(NW, EPT))
    deg = degp.T

    h = x
    for l in range(L):
        (acc,) = scl(h, pk3)
        h = _tc_dense(acc, deg, h,
                      W_self[l], W_neigh[l],
                      b[l].reshape(1, D),
                      ln_w[l].reshape(1, D),
                      ln_b[l].reshape(1, D))
    return h
